# baseline (speedup 1.0000x reference)
import dataclasses

import jax
import jax.numpy as jnp
from jax import lax
from jax.experimental import pallas as pl
from jax.experimental.pallas import tpu as pltpu
from jax.experimental.pallas import tpu_sc as plsc

B = 16384
F = 16
NC = 2
NS = 16
NW = NC * NS
BPW = B // NW
CH = 128
NCH = BPW // CH
GROUPS = BPW // F
M0 = 1000


def _fm_body(uix, iix, bix, aixf, uet, iet, m1t, m0t, lu, li, la0, lb1, out,
             uiv, iiv, biv, aiv, ubuf, ibuf, m1buf, m0v, lm0v,
             lub, lib, lb1b, ov, sem):
    wid = lax.axis_index("s") * NC + lax.axis_index("c")
    base = wid * BPW

    stage = [
        pltpu.async_copy(uix.at[wid], uiv, sem),
        pltpu.async_copy(iix.at[wid], iiv, sem),
        pltpu.async_copy(bix.at[wid], biv, sem),
        pltpu.async_copy(aixf.at[wid], aiv, sem),
        pltpu.async_copy(m0t, m0v, sem),
        pltpu.async_copy(la0, lm0v, sem),
    ]
    for c in stage[:3]:
        c.wait()

    copies = []
    for j in range(NCH):
        sl = pl.ds(j * CH, CH)
        for f in range(F):
            copies.append(
                pltpu.async_copy(uet.at[f].at[uiv.at[j]], ubuf.at[f, sl], sem))
            copies.append(
                pltpu.async_copy(iet.at[f].at[iiv.at[j]], ibuf.at[f, sl], sem))
            copies.append(
                pltpu.async_copy(m1t.at[f].at[biv.at[j]], m1buf.at[f, sl], sem))
        copies.append(pltpu.async_copy(lu.at[uiv.at[j]], lub.at[sl], sem))
        copies.append(pltpu.async_copy(li.at[iiv.at[j]], lib.at[sl], sem))
        copies.append(pltpu.async_copy(lb1.at[biv.at[j]], lb1b.at[sl], sem))
    for c in stage[3:]:
        c.wait()
    for c in copies:
        c.wait()

    @pl.loop(0, GROUPS)
    def _(g):
        row0 = g * F
        ridx = aiv[pl.ds(row0, F)]
        lm016 = plsc.load_gather(lm0v, [ridx])
        acc = lub[pl.ds(row0, F)] + lib[pl.ds(row0, F)] + lb1b[pl.ds(row0, F)] + lm016
        pw = jnp.zeros((F,), jnp.float32)
        for f in range(F):
            uf = ubuf[f, pl.ds(row0, F)]
            if_ = ibuf[f, pl.ds(row0, F)]
            bf = m1buf[f, pl.ds(row0, F)]
            af = plsc.load_gather(m0v, [jnp.full((F,), f, jnp.int32), ridx])
            s = uf + if_ + af + bf
            pw = pw + (s * s - uf * uf - if_ * if_ - af * af - bf * bf)
        tot = acc + 0.5 * pw
        ov[pl.ds(row0, F)] = 1.0 / (1.0 + jnp.exp(-tot))

    pltpu.sync_copy(ov, out.at[pl.ds(base, BPW)])


def kernel(user, item, metadata, user_emb, item_emb, meta_emb0, meta_emb1,
           lin_user, lin_item, lin_meta0, lin_meta1):
    mesh = plsc.VectorSubcoreMesh(core_axis_name="c", subcore_axis_name="s")
    cp = pltpu.CompilerParams()
    fields = pltpu.CompilerParams.__dataclass_fields__
    if "needs_layout_passes" in fields:
        cp = dataclasses.replace(cp, needs_layout_passes=False)
    if "use_tc_tiling_on_sc" in fields:
        cp = dataclasses.replace(cp, use_tc_tiling_on_sc=False)
    fm = pl.kernel(
        _fm_body,
        out_type=jax.ShapeDtypeStruct((B,), jnp.float32),
        mesh=mesh,
        compiler_params=cp,
        scratch_types=[
            pltpu.VMEM((NCH, CH), jnp.int32),
            pltpu.VMEM((NCH, CH), jnp.int32),
            pltpu.VMEM((NCH, CH), jnp.int32),
            pltpu.VMEM((BPW,), jnp.int32),
            pltpu.VMEM((F, BPW), jnp.float32),
            pltpu.VMEM((F, BPW), jnp.float32),
            pltpu.VMEM((F, BPW), jnp.float32),
            pltpu.VMEM((F, M0), jnp.float32),
            pltpu.VMEM((M0,), jnp.float32),
            pltpu.VMEM((BPW,), jnp.float32),
            pltpu.VMEM((BPW,), jnp.float32),
            pltpu.VMEM((BPW,), jnp.float32),
            pltpu.VMEM((BPW,), jnp.float32),
            pltpu.SemaphoreType.DMA,
        ],
    )
    uix = user.astype(jnp.int32).reshape(NW, NCH, CH)
    iix = item.astype(jnp.int32).reshape(NW, NCH, CH)
    bix = metadata[:, 1].astype(jnp.int32).reshape(NW, NCH, CH)
    aixf = metadata[:, 0].astype(jnp.int32).reshape(NW, BPW)
    return fm(uix, iix, bix, aixf,
              user_emb.T, item_emb.T, meta_emb1.T, meta_emb0.T,
              lin_user.reshape(-1), lin_item.reshape(-1),
              lin_meta0.reshape(-1), lin_meta1.reshape(-1))


# speedup vs baseline: 3.1699x; 3.1699x over previous
import dataclasses

import jax
import jax.numpy as jnp
from jax import lax
from jax.experimental import pallas as pl
from jax.experimental.pallas import tpu as pltpu
from jax.experimental.pallas import tpu_sc as plsc

B = 16384
F = 16
NC = 2
NS = 16
NW = NC * NS
BPW = B // NW
CH = 128
NCH = BPW // CH
GROUPS = BPW // F


def _fm_body(user, item, mt, ue, ie, ae, be, lu, li, la, lb, out,
             uiv, iiv, aiv, biv, usv, isv, asv, bsv,
             ur, ir, ar, br, lur, lir, lar, lbr, dtmp, ov, sem):
    wid = lax.axis_index("s") * NC + lax.axis_index("c")
    base = wid * BPW

    idx_copies = [
        pltpu.async_copy(user.at[pl.ds(base, BPW)], uiv, sem),
        pltpu.async_copy(item.at[pl.ds(base, BPW)], iiv, sem),
        pltpu.async_copy(mt.at[0, pl.ds(base, BPW)], aiv, sem),
        pltpu.async_copy(mt.at[1, pl.ds(base, BPW)], biv, sem),
    ]
    for c in idx_copies:
        c.wait()

    @pl.loop(0, BPW, step=F)
    def _(k):
        sl = pl.ds(k, F)
        usv[sl] = lax.shift_right_logical(uiv[sl], 3)
        isv[sl] = lax.shift_right_logical(iiv[sl], 3)
        asv[sl] = lax.shift_right_logical(aiv[sl], 3)
        bsv[sl] = lax.shift_right_logical(biv[sl], 3)

    copies = []
    for j in range(NCH):
        sl = pl.ds(j * CH, CH)
        copies.append(pltpu.async_copy(ue.at[uiv.at[sl]], ur.at[sl], sem))
        copies.append(pltpu.async_copy(ie.at[iiv.at[sl]], ir.at[sl], sem))
        copies.append(pltpu.async_copy(ae.at[aiv.at[sl]], ar.at[sl], sem))
        copies.append(pltpu.async_copy(be.at[biv.at[sl]], br.at[sl], sem))
        copies.append(pltpu.async_copy(lu.at[usv.at[sl]], lur.at[sl], sem))
        copies.append(pltpu.async_copy(li.at[isv.at[sl]], lir.at[sl], sem))
        copies.append(pltpu.async_copy(la.at[asv.at[sl]], lar.at[sl], sem))
        copies.append(pltpu.async_copy(lb.at[bsv.at[sl]], lbr.at[sl], sem))
    for c in copies:
        c.wait()

    lane = lax.broadcasted_iota(jnp.int32, (F,), 0)
    seven = jnp.full((F,), 7, jnp.int32)

    @pl.loop(0, GROUPS)
    def _(g):
        row0 = g * F
        for r in range(F):
            u = ur[row0 + r, :]
            i = ir[row0 + r, :]
            a = ar[row0 + r, :]
            b = br[row0 + r, :]
            s = u + i + a + b
            dtmp[r, :] = s * s - u * u - i * i - a * a - b * b
        pw = jnp.zeros((F,), jnp.float32)
        for j in range(F):
            pw = pw + plsc.load_gather(dtmp, [lane, jnp.full((F,), j, jnp.int32)])
        rows = lane + row0
        acc = (plsc.load_gather(lur, [rows, uiv[pl.ds(row0, F)] & seven])
               + plsc.load_gather(lir, [rows, iiv[pl.ds(row0, F)] & seven])
               + plsc.load_gather(lar, [rows, aiv[pl.ds(row0, F)] & seven])
               + plsc.load_gather(lbr, [rows, biv[pl.ds(row0, F)] & seven]))
        tot = acc + 0.5 * pw
        ov[pl.ds(row0, F)] = 1.0 / (1.0 + jnp.exp(-tot))

    pltpu.sync_copy(ov, out.at[pl.ds(base, BPW)])


def kernel(user, item, metadata, user_emb, item_emb, meta_emb0, meta_emb1,
           lin_user, lin_item, lin_meta0, lin_meta1):
    mesh = plsc.VectorSubcoreMesh(core_axis_name="c", subcore_axis_name="s")
    cp = pltpu.CompilerParams()
    fields = pltpu.CompilerParams.__dataclass_fields__
    if "needs_layout_passes" in fields:
        cp = dataclasses.replace(cp, needs_layout_passes=False)
    if "use_tc_tiling_on_sc" in fields:
        cp = dataclasses.replace(cp, use_tc_tiling_on_sc=False)
    fm = pl.kernel(
        _fm_body,
        out_type=jax.ShapeDtypeStruct((B,), jnp.float32),
        mesh=mesh,
        compiler_params=cp,
        scratch_types=[
            pltpu.VMEM((BPW,), jnp.int32),
            pltpu.VMEM((BPW,), jnp.int32),
            pltpu.VMEM((BPW,), jnp.int32),
            pltpu.VMEM((BPW,), jnp.int32),
            pltpu.VMEM((BPW,), jnp.int32),
            pltpu.VMEM((BPW,), jnp.int32),
            pltpu.VMEM((BPW,), jnp.int32),
            pltpu.VMEM((BPW,), jnp.int32),
            pltpu.VMEM((BPW, F), jnp.float32),
            pltpu.VMEM((BPW, F), jnp.float32),
            pltpu.VMEM((BPW, F), jnp.float32),
            pltpu.VMEM((BPW, F), jnp.float32),
            pltpu.VMEM((BPW, 8), jnp.float32),
            pltpu.VMEM((BPW, 8), jnp.float32),
            pltpu.VMEM((BPW, 8), jnp.float32),
            pltpu.VMEM((BPW, 8), jnp.float32),
            pltpu.VMEM((F, F), jnp.float32),
            pltpu.VMEM((BPW,), jnp.float32),
            pltpu.SemaphoreType.DMA,
        ],
    )
    return fm(user.astype(jnp.int32), item.astype(jnp.int32),
              metadata.T.astype(jnp.int32),
              user_emb, item_emb, meta_emb0, meta_emb1,
              lin_user.reshape(-1, 8), lin_item.reshape(-1, 8),
              lin_meta0.reshape(-1, 8), lin_meta1.reshape(-1, 8))


# speedup vs baseline: 7.8116x; 2.4643x over previous
import dataclasses

import jax
import jax.numpy as jnp
from jax import lax
from jax.experimental import pallas as pl
from jax.experimental.pallas import tpu as pltpu
from jax.experimental.pallas import tpu_sc as plsc

B = 16384
F = 16
NC = 2
NS = 16
NW = NC * NS
BPW = B // NW
GROUPS = BPW // F
NU = 1000000
NM0 = 1000
NM1 = 100000
W = 128


def _fm_body(user, item, mt, ue, ie, ae, be, lu, li, la0, lb1, out,
             uiv, iiv, aiv, biv, ubv, ibv, bbv, urm, irm, brm,
             uwin, iwin, bwin, luw, liw, lbw,
             m0v, lm0v, ov, sem):
    cid = lax.axis_index("c")
    sid = lax.axis_index("s")
    wid = sid * NC + cid
    base = pl.multiple_of(wid * BPW, BPW)

    idx_copies = [
        pltpu.async_copy(user.at[pl.ds(base, BPW)], uiv, sem),
        pltpu.async_copy(item.at[pl.ds(base, BPW)], iiv, sem),
        pltpu.async_copy(mt.at[0, pl.ds(base, BPW)], aiv, sem),
        pltpu.async_copy(mt.at[1, pl.ds(base, BPW)], biv, sem),
        pltpu.async_copy(ae, m0v, sem),
        pltpu.async_copy(la0, lm0v, sem),
    ]
    for c in idx_copies[:4]:
        c.wait()

    @pl.loop(0, BPW, step=F)
    def _(k):
        sl = pl.ds(k, F)
        ubv[sl] = uiv[sl] & -W
        ibv[sl] = iiv[sl] & -W
        bbv[sl] = biv[sl] & -W
        urm[sl] = uiv[sl] & (W - 1)
        irm[sl] = iiv[sl] & (W - 1)
        brm[sl] = biv[sl] & (W - 1)

    for c in idx_copies[4:]:
        c.wait()

    lane = lax.broadcasted_iota(jnp.int32, (F,), 0)

    @pl.loop(0, GROUPS)
    def _(g):
        row0 = g * F
        ubs = ubv[pl.ds(row0, F)]
        ibs = ibv[pl.ds(row0, F)]
        bbs = bbv[pl.ds(row0, F)]
        for r0 in range(0, F, 4):
            wcopies = []
            for r in range(r0, r0 + 4):
                ub = pl.multiple_of(ubs[r], W)
                ib = pl.multiple_of(ibs[r], W)
                bb = pl.multiple_of(bbs[r], W)
                wcopies.append(pltpu.async_copy(ue.at[:, pl.ds(ub, W)], uwin.at[r], sem))
                wcopies.append(pltpu.async_copy(ie.at[:, pl.ds(ib, W)], iwin.at[r], sem))
                wcopies.append(pltpu.async_copy(be.at[:, pl.ds(bb, W)], bwin.at[r], sem))
                wcopies.append(pltpu.async_copy(lu.at[pl.ds(ub, W)], luw.at[r], sem))
                wcopies.append(pltpu.async_copy(li.at[pl.ds(ib, W)], liw.at[r], sem))
                wcopies.append(pltpu.async_copy(lb1.at[pl.ds(bb, W)], lbw.at[r], sem))
            for c in wcopies:
                c.wait()

        urm16 = urm[pl.ds(row0, F)]
        irm16 = irm[pl.ds(row0, F)]
        brm16 = brm[pl.ds(row0, F)]
        aidx16 = aiv[pl.ds(row0, F)]
        acc = (plsc.load_gather(luw, [lane, urm16])
               + plsc.load_gather(liw, [lane, irm16])
               + plsc.load_gather(lbw, [lane, brm16])
               + plsc.load_gather(lm0v, [aidx16]))
        pw = jnp.zeros((F,), jnp.float32)
        for f in range(F):
            fv = jnp.full((F,), f, jnp.int32)
            uf = plsc.load_gather(uwin, [lane, fv, urm16])
            if_ = plsc.load_gather(iwin, [lane, fv, irm16])
            af = plsc.load_gather(m0v, [fv, aidx16])
            bf = plsc.load_gather(bwin, [lane, fv, brm16])
            s = uf + if_ + af + bf
            pw = pw + (s * s - uf * uf - if_ * if_ - af * af - bf * bf)
        tot = acc + 0.5 * pw
        ov[pl.ds(row0, F)] = 1.0 / (1.0 + jnp.exp(-tot))

    pltpu.sync_copy(ov, out.at[pl.ds(base, BPW)])


def kernel(user, item, metadata, user_emb, item_emb, meta_emb0, meta_emb1,
           lin_user, lin_item, lin_meta0, lin_meta1):
    mesh = plsc.VectorSubcoreMesh(core_axis_name="c", subcore_axis_name="s")
    cp = pltpu.CompilerParams()
    fields = pltpu.CompilerParams.__dataclass_fields__
    if "needs_layout_passes" in fields:
        cp = dataclasses.replace(cp, needs_layout_passes=False)
    fm = pl.kernel(
        _fm_body,
        out_type=jax.ShapeDtypeStruct((B,), jnp.float32),
        mesh=mesh,
        compiler_params=cp,
        scratch_types=[
            pltpu.VMEM((BPW,), jnp.int32),
            pltpu.VMEM((BPW,), jnp.int32),
            pltpu.VMEM((BPW,), jnp.int32),
            pltpu.VMEM((BPW,), jnp.int32),
            pltpu.VMEM((BPW,), jnp.int32),
            pltpu.VMEM((BPW,), jnp.int32),
            pltpu.VMEM((BPW,), jnp.int32),
            pltpu.VMEM((BPW,), jnp.int32),
            pltpu.VMEM((BPW,), jnp.int32),
            pltpu.VMEM((BPW,), jnp.int32),
            pltpu.VMEM((F, F, W), jnp.float32),
            pltpu.VMEM((F, F, W), jnp.float32),
            pltpu.VMEM((F, F, W), jnp.float32),
            pltpu.VMEM((F, W), jnp.float32),
            pltpu.VMEM((F, W), jnp.float32),
            pltpu.VMEM((F, W), jnp.float32),
            pltpu.VMEM((F, NM0), jnp.float32),
            pltpu.VMEM((NM0,), jnp.float32),
            pltpu.VMEM((BPW,), jnp.float32),
            pltpu.SemaphoreType.DMA,
        ],
    )
    return fm(user.astype(jnp.int32), item.astype(jnp.int32),
              metadata.T.astype(jnp.int32),
              user_emb.T, item_emb.T, meta_emb0.T, meta_emb1.T,
              lin_user.reshape(-1), lin_item.reshape(-1),
              lin_meta0.reshape(-1), lin_meta1.reshape(-1))


# speedup vs baseline: 9.7884x; 1.2531x over previous
import dataclasses

import jax
import jax.numpy as jnp
from jax import lax
from jax.experimental import pallas as pl
from jax.experimental.pallas import tpu as pltpu
from jax.experimental.pallas import tpu_sc as plsc

B = 16384
F = 16
NC = 2
NS = 16
NW = NC * NS
BPW = B // NW
GROUPS = BPW // F
NU = 1000000
NM0 = 1000
NM1 = 100000
W = 128


def _fm_body(user, item, mt, ue, ie, ae, be, lu, li, la0, lb1, out,
             uiv, iiv, aiv, biv, ubv, ibv, bbv, urm, irm, brm,
             uwin, iwin, bwin, luw, liw, lbw,
             m0v, lm0v, ov, sem):
    cid = lax.axis_index("c")
    sid = lax.axis_index("s")
    wid = sid * NC + cid
    base = pl.multiple_of(wid * BPW, BPW)

    idx_copies = [
        pltpu.async_copy(user.at[pl.ds(base, BPW)], uiv, sem),
        pltpu.async_copy(item.at[pl.ds(base, BPW)], iiv, sem),
        pltpu.async_copy(mt.at[0, pl.ds(base, BPW)], aiv, sem),
        pltpu.async_copy(mt.at[1, pl.ds(base, BPW)], biv, sem),
        pltpu.async_copy(ae, m0v, sem),
        pltpu.async_copy(la0, lm0v, sem),
    ]
    for c in idx_copies[:4]:
        c.wait()

    @pl.loop(0, BPW, step=F)
    def _(k):
        sl = pl.ds(k, F)
        ubv[sl] = uiv[sl] & -W
        ibv[sl] = iiv[sl] & -W
        bbv[sl] = biv[sl] & -W
        urm[sl] = uiv[sl] & (W - 1)
        irm[sl] = iiv[sl] & (W - 1)
        brm[sl] = biv[sl] & (W - 1)

    for c in idx_copies[4:]:
        c.wait()

    lane = lax.broadcasted_iota(jnp.int32, (F,), 0)

    @pl.loop(0, GROUPS)
    def _(g):
        row0 = g * F
        ubs = ubv[pl.ds(row0, F)]
        ibs = ibv[pl.ds(row0, F)]
        bbs = bbv[pl.ds(row0, F)]
        def _fire(r0):
            chunk = []
            for r in range(r0, r0 + 4):
                ub = pl.multiple_of(ubs[r], W)
                ib = pl.multiple_of(ibs[r], W)
                bb = pl.multiple_of(bbs[r], W)
                chunk.append(pltpu.async_copy(ue.at[:, pl.ds(ub, W)], uwin.at[r], sem))
                chunk.append(pltpu.async_copy(ie.at[:, pl.ds(ib, W)], iwin.at[r], sem))
                chunk.append(pltpu.async_copy(be.at[:, pl.ds(bb, W)], bwin.at[r], sem))
                chunk.append(pltpu.async_copy(lu.at[pl.ds(ub, W)], luw.at[r], sem))
                chunk.append(pltpu.async_copy(li.at[pl.ds(ib, W)], liw.at[r], sem))
                chunk.append(pltpu.async_copy(lb1.at[pl.ds(bb, W)], lbw.at[r], sem))
            return chunk

        pending = _fire(0)
        for r0 in range(4, F, 4):
            nxt = _fire(r0)
            for c in pending:
                c.wait()
            pending = nxt
        for c in pending:
            c.wait()

        urm16 = urm[pl.ds(row0, F)]
        irm16 = irm[pl.ds(row0, F)]
        brm16 = brm[pl.ds(row0, F)]
        aidx16 = aiv[pl.ds(row0, F)]
        acc = (plsc.load_gather(luw, [lane, urm16])
               + plsc.load_gather(liw, [lane, irm16])
               + plsc.load_gather(lbw, [lane, brm16])
               + plsc.load_gather(lm0v, [aidx16]))
        pw = jnp.zeros((F,), jnp.float32)
        for f in range(F):
            fv = jnp.full((F,), f, jnp.int32)
            uf = plsc.load_gather(uwin, [lane, fv, urm16])
            if_ = plsc.load_gather(iwin, [lane, fv, irm16])
            af = plsc.load_gather(m0v, [fv, aidx16])
            bf = plsc.load_gather(bwin, [lane, fv, brm16])
            s = uf + if_ + af + bf
            pw = pw + (s * s - uf * uf - if_ * if_ - af * af - bf * bf)
        tot = acc + 0.5 * pw
        ov[pl.ds(row0, F)] = 1.0 / (1.0 + jnp.exp(-tot))

    pltpu.sync_copy(ov, out.at[pl.ds(base, BPW)])


def kernel(user, item, metadata, user_emb, item_emb, meta_emb0, meta_emb1,
           lin_user, lin_item, lin_meta0, lin_meta1):
    mesh = plsc.VectorSubcoreMesh(core_axis_name="c", subcore_axis_name="s")
    cp = pltpu.CompilerParams()
    fields = pltpu.CompilerParams.__dataclass_fields__
    if "needs_layout_passes" in fields:
        cp = dataclasses.replace(cp, needs_layout_passes=False)
    fm = pl.kernel(
        _fm_body,
        out_type=jax.ShapeDtypeStruct((B,), jnp.float32),
        mesh=mesh,
        compiler_params=cp,
        scratch_types=[
            pltpu.VMEM((BPW,), jnp.int32),
            pltpu.VMEM((BPW,), jnp.int32),
            pltpu.VMEM((BPW,), jnp.int32),
            pltpu.VMEM((BPW,), jnp.int32),
            pltpu.VMEM((BPW,), jnp.int32),
            pltpu.VMEM((BPW,), jnp.int32),
            pltpu.VMEM((BPW,), jnp.int32),
            pltpu.VMEM((BPW,), jnp.int32),
            pltpu.VMEM((BPW,), jnp.int32),
            pltpu.VMEM((BPW,), jnp.int32),
            pltpu.VMEM((F, F, W), jnp.float32),
            pltpu.VMEM((F, F, W), jnp.float32),
            pltpu.VMEM((F, F, W), jnp.float32),
            pltpu.VMEM((F, W), jnp.float32),
            pltpu.VMEM((F, W), jnp.float32),
            pltpu.VMEM((F, W), jnp.float32),
            pltpu.VMEM((F, NM0), jnp.float32),
            pltpu.VMEM((NM0,), jnp.float32),
            pltpu.VMEM((BPW,), jnp.float32),
            pltpu.SemaphoreType.DMA,
        ],
    )
    return fm(user.astype(jnp.int32), item.astype(jnp.int32),
              metadata.T.astype(jnp.int32),
              user_emb.T, item_emb.T, meta_emb0.T, meta_emb1.T,
              lin_user.reshape(-1), lin_item.reshape(-1),
              lin_meta0.reshape(-1), lin_meta1.reshape(-1))


# speedup vs baseline: 9.8367x; 1.0049x over previous
import dataclasses

import jax
import jax.numpy as jnp
from jax import lax
from jax.experimental import pallas as pl
from jax.experimental.pallas import tpu as pltpu
from jax.experimental.pallas import tpu_sc as plsc

B = 16384
F = 16
NC = 2
NS = 16
NW = NC * NS
BPW = B // NW
GROUPS = BPW // F
NU = 1000000
NM0 = 1000
NM1 = 100000
W = 128


def _fm_body(user, item, mt, ue, ie, ae, be, lu, li, la0, lb1, out,
             uiv, iiv, aiv, biv, ubv, ibv, bbv, urm, irm, brm,
             uwin, iwin, bwin, luw, liw, lbw,
             m0v, lm0v, ov, sem):
    cid = lax.axis_index("c")
    sid = lax.axis_index("s")
    wid = sid * NC + cid
    base = pl.multiple_of(wid * BPW, BPW)

    idx_copies = [
        pltpu.async_copy(user.at[pl.ds(base, BPW)], uiv, sem),
        pltpu.async_copy(item.at[pl.ds(base, BPW)], iiv, sem),
        pltpu.async_copy(mt.at[0, pl.ds(base, BPW)], aiv, sem),
        pltpu.async_copy(mt.at[1, pl.ds(base, BPW)], biv, sem),
        pltpu.async_copy(ae, m0v, sem),
        pltpu.async_copy(la0, lm0v, sem),
    ]
    for c in idx_copies[:4]:
        c.wait()

    @pl.loop(0, BPW, step=F)
    def _(k):
        sl = pl.ds(k, F)
        ubv[sl] = uiv[sl] & -W
        ibv[sl] = iiv[sl] & -W
        bbv[sl] = biv[sl] & -W
        urm[sl] = uiv[sl] & (W - 1)
        irm[sl] = iiv[sl] & (W - 1)
        brm[sl] = biv[sl] & (W - 1)

    for c in idx_copies[4:]:
        c.wait()

    lane = lax.broadcasted_iota(jnp.int32, (F,), 0)

    @pl.loop(0, GROUPS)
    def _(g):
        row0 = g * F
        ubs = ubv[pl.ds(row0, F)]
        ibs = ibv[pl.ds(row0, F)]
        bbs = bbv[pl.ds(row0, F)]
        def _fire(r0):
            chunk = []
            for r in range(r0, r0 + 4):
                ub = pl.multiple_of(ubs[r], W)
                ib = pl.multiple_of(ibs[r], W)
                bb = pl.multiple_of(bbs[r], W)
                chunk.append(pltpu.async_copy(ue.at[:, pl.ds(ub, W)], uwin.at[r], sem))
                chunk.append(pltpu.async_copy(ie.at[:, pl.ds(ib, W)], iwin.at[r], sem))
                chunk.append(pltpu.async_copy(be.at[:, pl.ds(bb, W)], bwin.at[r], sem))
                chunk.append(pltpu.async_copy(lu.at[pl.ds(ub, W)], luw.at[r], sem))
                chunk.append(pltpu.async_copy(li.at[pl.ds(ib, W)], liw.at[r], sem))
                chunk.append(pltpu.async_copy(lb1.at[pl.ds(bb, W)], lbw.at[r], sem))
            return chunk

        chunks = [_fire(0), _fire(4)]
        for r0 in range(8, F, 4):
            chunks.append(_fire(r0))
            for c in chunks.pop(0):
                c.wait()
        for ch in chunks:
            for c in ch:
                c.wait()

        urm16 = urm[pl.ds(row0, F)]
        irm16 = irm[pl.ds(row0, F)]
        brm16 = brm[pl.ds(row0, F)]
        aidx16 = aiv[pl.ds(row0, F)]
        acc = (plsc.load_gather(luw, [lane, urm16])
               + plsc.load_gather(liw, [lane, irm16])
               + plsc.load_gather(lbw, [lane, brm16])
               + plsc.load_gather(lm0v, [aidx16]))
        pw = jnp.zeros((F,), jnp.float32)
        for f in range(F):
            fv = jnp.full((F,), f, jnp.int32)
            uf = plsc.load_gather(uwin, [lane, fv, urm16])
            if_ = plsc.load_gather(iwin, [lane, fv, irm16])
            af = plsc.load_gather(m0v, [fv, aidx16])
            bf = plsc.load_gather(bwin, [lane, fv, brm16])
            s = uf + if_ + af + bf
            pw = pw + (s * s - uf * uf - if_ * if_ - af * af - bf * bf)
        tot = acc + 0.5 * pw
        ov[pl.ds(row0, F)] = 1.0 / (1.0 + jnp.exp(-tot))

    pltpu.sync_copy(ov, out.at[pl.ds(base, BPW)])


def kernel(user, item, metadata, user_emb, item_emb, meta_emb0, meta_emb1,
           lin_user, lin_item, lin_meta0, lin_meta1):
    mesh = plsc.VectorSubcoreMesh(core_axis_name="c", subcore_axis_name="s")
    cp = pltpu.CompilerParams()
    fields = pltpu.CompilerParams.__dataclass_fields__
    if "needs_layout_passes" in fields:
        cp = dataclasses.replace(cp, needs_layout_passes=False)
    fm = pl.kernel(
        _fm_body,
        out_type=jax.ShapeDtypeStruct((B,), jnp.float32),
        mesh=mesh,
        compiler_params=cp,
        scratch_types=[
            pltpu.VMEM((BPW,), jnp.int32),
            pltpu.VMEM((BPW,), jnp.int32),
            pltpu.VMEM((BPW,), jnp.int32),
            pltpu.VMEM((BPW,), jnp.int32),
            pltpu.VMEM((BPW,), jnp.int32),
            pltpu.VMEM((BPW,), jnp.int32),
            pltpu.VMEM((BPW,), jnp.int32),
            pltpu.VMEM((BPW,), jnp.int32),
            pltpu.VMEM((BPW,), jnp.int32),
            pltpu.VMEM((BPW,), jnp.int32),
            pltpu.VMEM((F, F, W), jnp.float32),
            pltpu.VMEM((F, F, W), jnp.float32),
            pltpu.VMEM((F, F, W), jnp.float32),
            pltpu.VMEM((F, W), jnp.float32),
            pltpu.VMEM((F, W), jnp.float32),
            pltpu.VMEM((F, W), jnp.float32),
            pltpu.VMEM((F, NM0), jnp.float32),
            pltpu.VMEM((NM0,), jnp.float32),
            pltpu.VMEM((BPW,), jnp.float32),
            pltpu.SemaphoreType.DMA,
        ],
    )
    return fm(user.astype(jnp.int32), item.astype(jnp.int32),
              metadata.T.astype(jnp.int32),
              user_emb.T, item_emb.T, meta_emb0.T, meta_emb1.T,
              lin_user.reshape(-1), lin_item.reshape(-1),
              lin_meta0.reshape(-1), lin_meta1.reshape(-1))
